# baseline (speedup 1.0000x reference)
import functools

import jax
import jax.numpy as jnp
from jax import lax
from jax.experimental import pallas as pl
from jax.experimental.pallas import tpu as pltpu
from jax.experimental.pallas import tpu_sc as plsc

N = 10000
D = 128
NPAD = 10240
PAD_ROW = NPAD - 1
NC = 2
NS = 16
NW = NC * NS
CH = 128
ROWS_PER_TILE = NPAD // NS
BLK = 512
SPLIT0 = 0.9


def _cdiv(a, b):
    return (a + b - 1) // b


def _sc_mesh():
    return plsc.VectorSubcoreMesh(core_axis_name="c", subcore_axis_name="s")


@functools.partial(jax.jit, static_argnames=("cpw",))
def _sc_degree(dst2d, zerosf, onesf, *, cpw):
    @functools.partial(
        pl.kernel,
        out_type=jax.ShapeDtypeStruct((NC, NPAD, D), jnp.float32),
        mesh=_sc_mesh(),
        scratch_types=[
            pltpu.VMEM((cpw, CH), jnp.int32),
            pltpu.VMEM((CH, D), jnp.float32),
            pltpu.VMEM_SHARED((NPAD, D), jnp.float32),
        ],
    )
    def deg_kernel(dst_hbm, z_hbm, ones_hbm, out_hbm, dsts_v, ones_v, acc_sh):
        c = lax.axis_index("c")
        s = lax.axis_index("s")
        wid = c * NS + s
        r0 = s * ROWS_PER_TILE
        pltpu.sync_copy(dst_hbm.at[pl.ds(wid * cpw, cpw)], dsts_v)
        pltpu.sync_copy(z_hbm.at[pl.ds(r0, ROWS_PER_TILE)],
                        acc_sh.at[pl.ds(r0, ROWS_PER_TILE)])
        pltpu.sync_copy(ones_hbm, ones_v)
        plsc.subcore_barrier()

        def body(j, carry):
            pltpu.sync_copy(ones_v, acc_sh.at[dsts_v.at[j]], add=True)
            return carry

        lax.fori_loop(0, cpw, body, 0)
        plsc.subcore_barrier()
        pltpu.sync_copy(acc_sh.at[pl.ds(r0, ROWS_PER_TILE)],
                        out_hbm.at[c, pl.ds(r0, ROWS_PER_TILE)])

    return deg_kernel(dst2d, zerosf, onesf)


@functools.partial(jax.jit, static_argnames=("cpw0", "cpw1"))
def _sc_scatter(packed2d, g, zerosf, *, cpw0, cpw1):
    @functools.partial(
        pl.kernel,
        out_type=jax.ShapeDtypeStruct((NC, NPAD, D), jnp.float32),
        mesh=_sc_mesh(),
        scratch_types=[
            pltpu.VMEM((max(cpw0, cpw1) // 2, CH), jnp.int32),
            pltpu.VMEM((CH,), jnp.int32),
            pltpu.VMEM((CH,), jnp.int32),
            pltpu.VMEM((CH,), jnp.int32),
            pltpu.VMEM((CH,), jnp.int32),
            pltpu.VMEM((CH, D), jnp.float32),
            pltpu.VMEM((CH, D), jnp.float32),
            pltpu.VMEM_SHARED((NPAD, D), jnp.float32),
            pltpu.SemaphoreType.DMA,
            pltpu.SemaphoreType.DMA,
        ],
    )
    def msg_kernel(packed_hbm, g_hbm, z_hbm, out_hbm,
                   idx_v, src0, src1, dst0, dst1, buf0, buf1, acc_sh,
                   sem0, sem1):
        c = lax.axis_index("c")
        s = lax.axis_index("s")
        r0 = s * ROWS_PER_TILE
        bufs = (buf0, buf1)
        srcb = (src0, src1)
        dstb = (dst0, dst1)
        sems = (sem0, sem1)
        nch = jnp.where(c == 0, cpw0, cpw1)
        hch = nch // 2
        row0 = jnp.where(c == 0, s * cpw0, NS * cpw0 + s * cpw1)
        maxh = max(cpw0, cpw1) // 2

        def unpack(j, b):
            for k in range(CH // 16):
                p = idx_v[j, pl.ds(k * 16, 16)]
                srcb[b][pl.ds(k * 16, 16)] = p & 0x3FFF
                dstb[b][pl.ds(k * 16, 16)] = p >> 14

        pltpu.sync_copy(z_hbm.at[pl.ds(r0, ROWS_PER_TILE)],
                        acc_sh.at[pl.ds(r0, ROWS_PER_TILE)])
        plsc.subcore_barrier()

        def chunk(j, b):
            unpack(j, b)
            pltpu.async_copy(g_hbm.at[srcb[b]], bufs[b], sems[b]).wait()
            pltpu.sync_copy(bufs[b], acc_sh.at[dstb[b]], add=True)

        def body(t, carry):
            chunk(2 * t, 0)
            chunk(2 * t + 1, 1)
            return carry

        for h in range(2):
            pltpu.sync_copy(
                packed_hbm.at[pl.ds(pl.multiple_of(row0 + h * hch, 8), maxh)],
                idx_v)
            lax.fori_loop(0, hch // 2, body, 0)
        plsc.subcore_barrier()
        pltpu.sync_copy(acc_sh.at[pl.ds(r0, ROWS_PER_TILE)],
                        out_hbm.at[c, pl.ds(r0, ROWS_PER_TILE)])

    return msg_kernel(packed2d, g, zerosf)


_HI = jax.lax.Precision.HIGHEST


def _dinv_g1_body(dega_ref, degb_ref, x_ref, w_ref, dinv_ref, g_ref):
    i = pl.program_id(0)
    deg = dega_ref[...][:, :1] + degb_ref[...][:, :1] + 1.0
    rows = i * BLK + lax.broadcasted_iota(jnp.int32, (BLK, 1), 0)
    dinv = jnp.where(rows < N, lax.rsqrt(deg), 0.0)
    dinvm = jnp.broadcast_to(dinv, (BLK, D))
    dinv_ref[...] = dinvm
    g_ref[...] = jnp.dot(x_ref[...], w_ref[...],
                         preferred_element_type=jnp.float32,
                         precision=_HI) * dinvm


@jax.jit
def _tc_dinv_g1(dega, degb, x_p, W1):
    grid = (NPAD // BLK,)
    blk = pl.BlockSpec((BLK, D), lambda i: (i, 0))
    return pl.pallas_call(
        _dinv_g1_body,
        grid=grid,
        in_specs=[blk, blk, blk, pl.BlockSpec((D, D), lambda i: (0, 0))],
        out_specs=[blk, blk],
        out_shape=[
            jax.ShapeDtypeStruct((NPAD, D), jnp.float32),
            jax.ShapeDtypeStruct((NPAD, D), jnp.float32),
        ],
    )(dega, degb, x_p, W1)


def _layer_body(acca_ref, accb_ref, g_ref, dinv_ref, b_ref, xprev_ref,
                wa_ref, wb_ref, xo_ref, go_ref):
    dm = dinv_ref[...]
    xn = jnp.maximum((acca_ref[...] + accb_ref[...] + g_ref[...]) * dm
                     + b_ref[...], 0.0)
    xo_ref[...] = xn
    go_ref[...] = (jnp.dot(xn, wa_ref[...],
                           preferred_element_type=jnp.float32, precision=_HI)
                   + jnp.dot(xprev_ref[...], wb_ref[...],
                             preferred_element_type=jnp.float32,
                             precision=_HI)) * dm


@jax.jit
def _tc_layer(acca, accb, g, dinvm, b, xprev, Wa, Wb):
    grid = (NPAD // BLK,)
    blk = pl.BlockSpec((BLK, D), lambda i: (i, 0))
    return pl.pallas_call(
        _layer_body,
        grid=grid,
        in_specs=[
            blk, blk, blk, blk,
            pl.BlockSpec((1, D), lambda i: (0, 0)),
            blk,
            pl.BlockSpec((D, D), lambda i: (0, 0)),
            pl.BlockSpec((D, D), lambda i: (0, 0)),
        ],
        out_specs=[blk, blk],
        out_shape=[
            jax.ShapeDtypeStruct((NPAD, D), jnp.float32),
            jax.ShapeDtypeStruct((NPAD, D), jnp.float32),
        ],
    )(acca, accb, g, dinvm, b, xprev, Wa, Wb)


def _final_body(acca_ref, accb_ref, g_ref, dinv_ref, b_ref, o_ref):
    o_ref[...] = ((acca_ref[...] + accb_ref[...] + g_ref[...])
                  * dinv_ref[...] + b_ref[...])


@jax.jit
def _tc_final(acca, accb, g, dinvm, b):
    grid = (NPAD // BLK,)
    blk = pl.BlockSpec((BLK, D), lambda i: (i, 0))
    return pl.pallas_call(
        _final_body,
        grid=grid,
        in_specs=[blk, blk, blk, blk, pl.BlockSpec((1, D), lambda i: (0, 0))],
        out_specs=blk,
        out_shape=jax.ShapeDtypeStruct((NPAD, D), jnp.float32),
    )(acca, accb, g, dinvm, b)


def kernel(x, edge_index, percent, ricci_curvature, W1, b1, W2, b2, W3, b3):
    del percent, ricci_curvature
    E = edge_index.shape[1]
    cpw = 2 * _cdiv(E, NW * CH * 2)
    EPAD = NW * cpw * CH
    total = NW * cpw
    cpw0 = 16 * int(round(total * SPLIT0 / NS / 16))
    cpw1 = total // NS - cpw0

    pad = jnp.full((EPAD - E,), PAD_ROW, dtype=edge_index.dtype)
    src_p = jnp.concatenate([edge_index[0], pad])
    dst_p = jnp.concatenate([edge_index[1], pad]).reshape(NW * cpw, CH)
    packed = (src_p + (dst_p.reshape(-1) << 14)).reshape(NW * cpw, CH)
    ptail = jnp.full((max(cpw0, cpw1), CH), PAD_ROW + (PAD_ROW << 14),
                     dtype=jnp.int32)
    packed2d = jnp.concatenate([packed, ptail], axis=0)
    x_p = jnp.pad(x, ((0, NPAD - N), (0, 0)))
    onesf = jnp.ones((CH, D), jnp.float32)
    zerosf = jnp.zeros((NPAD, D), jnp.float32)

    degp = _sc_degree(dst_p, zerosf, onesf, cpw=cpw)
    dinvm, g1 = _tc_dinv_g1(degp[0], degp[1], x_p, W1)

    acc1 = _sc_scatter(packed2d, g1, zerosf, cpw0=cpw0, cpw1=cpw1)
    x1, g2 = _tc_layer(acc1[0], acc1[1], g1, dinvm, b1.reshape(1, D),
                       x_p, W2[:D], W2[D:])

    acc2 = _sc_scatter(packed2d, g2, zerosf, cpw0=cpw0, cpw1=cpw1)
    x2, g3 = _tc_layer(acc2[0], acc2[1], g2, dinvm, b2.reshape(1, D),
                       x1, W3[:D], W3[D:])

    acc3 = _sc_scatter(packed2d, g3, zerosf, cpw0=cpw0, cpw1=cpw1)
    out = _tc_final(acc3[0], acc3[1], g3, dinvm, b3.reshape(1, D))

    return out[:N], x1[:N], x2[:N]


# speedup vs baseline: 1.0210x; 1.0210x over previous
import functools

import jax
import jax.numpy as jnp
from jax import lax
from jax.experimental import pallas as pl
from jax.experimental.pallas import tpu as pltpu
from jax.experimental.pallas import tpu_sc as plsc

N = 10000
D = 128
HD = 64
NPAD = 10240
PAD_ROW = NPAD - 1
NC = 2
NS = 16
NW = NC * NS
CH = 128
ROWS_PER_TILE = NPAD // NS
BLK = 512
SPLIT0 = 0.9


def _cdiv(a, b):
    return (a + b - 1) // b


def _sc_mesh():
    return plsc.VectorSubcoreMesh(core_axis_name="c", subcore_axis_name="s")


@functools.partial(jax.jit, static_argnames=("cpw",))
def _sc_degree(dst2d, zerosf, onesf, *, cpw):
    @functools.partial(
        pl.kernel,
        out_type=jax.ShapeDtypeStruct((NC, NPAD, HD), jnp.float32),
        mesh=_sc_mesh(),
        scratch_types=[
            pltpu.VMEM((cpw, CH), jnp.int32),
            pltpu.VMEM((CH, HD), jnp.float32),
            pltpu.VMEM_SHARED((NPAD, HD), jnp.float32),
        ],
    )
    def deg_kernel(dst_hbm, z_hbm, ones_hbm, out_hbm, dsts_v, ones_v, acc_sh):
        c = lax.axis_index("c")
        s = lax.axis_index("s")
        wid = c * NS + s
        r0 = s * ROWS_PER_TILE
        pltpu.sync_copy(dst_hbm.at[pl.ds(wid * cpw, cpw)], dsts_v)
        pltpu.sync_copy(z_hbm.at[pl.ds(r0, ROWS_PER_TILE)],
                        acc_sh.at[pl.ds(r0, ROWS_PER_TILE)])
        pltpu.sync_copy(ones_hbm, ones_v)
        plsc.subcore_barrier()

        def body(j, carry):
            pltpu.sync_copy(ones_v, acc_sh.at[dsts_v.at[j]], add=True)
            return carry

        lax.fori_loop(0, cpw, body, 0)
        plsc.subcore_barrier()
        pltpu.sync_copy(acc_sh.at[pl.ds(r0, ROWS_PER_TILE)],
                        out_hbm.at[c, pl.ds(r0, ROWS_PER_TILE)])

    return deg_kernel(dst2d, zerosf, onesf)


@functools.partial(jax.jit, static_argnames=("cpw0", "cpw1"))
def _sc_scatter(packed2d, g, zerosf, *, cpw0, cpw1):
    @functools.partial(
        pl.kernel,
        out_type=jax.ShapeDtypeStruct((NC, NPAD, D), jnp.float32),
        mesh=_sc_mesh(),
        scratch_types=[
            pltpu.VMEM((max(cpw0, cpw1) // 2, CH), jnp.int32),
            pltpu.VMEM((CH,), jnp.int32),
            pltpu.VMEM((CH,), jnp.int32),
            pltpu.VMEM((CH,), jnp.int32),
            pltpu.VMEM((CH,), jnp.int32),
            pltpu.VMEM((CH, D), jnp.float32),
            pltpu.VMEM((CH, D), jnp.float32),
            pltpu.VMEM_SHARED((NPAD, D), jnp.float32),
            pltpu.SemaphoreType.DMA,
            pltpu.SemaphoreType.DMA,
        ],
    )
    def msg_kernel(packed_hbm, g_hbm, z_hbm, out_hbm,
                   idx_v, src0, src1, dst0, dst1, buf0, buf1, acc_sh,
                   sem0, sem1):
        c = lax.axis_index("c")
        s = lax.axis_index("s")
        r0 = s * ROWS_PER_TILE
        bufs = (buf0, buf1)
        srcb = (src0, src1)
        dstb = (dst0, dst1)
        sems = (sem0, sem1)
        nch = jnp.where(c == 0, cpw0, cpw1)
        hch = nch // 2
        row0 = jnp.where(c == 0, s * cpw0, NS * cpw0 + s * cpw1)
        maxh = max(cpw0, cpw1) // 2

        def unpack(j, b):
            for k in range(CH // 16):
                p = idx_v[j, pl.ds(k * 16, 16)]
                srcb[b][pl.ds(k * 16, 16)] = p & 0x3FFF
                dstb[b][pl.ds(k * 16, 16)] = p >> 14

        pltpu.sync_copy(z_hbm.at[pl.ds(r0, ROWS_PER_TILE)],
                        acc_sh.at[pl.ds(r0, ROWS_PER_TILE)])
        plsc.subcore_barrier()

        def chunk(j, b):
            @pl.when(j + 1 < hch)
            def _():
                unpack(j + 1, 1 - b)
                pltpu.async_copy(g_hbm.at[srcb[1 - b]],
                                 bufs[1 - b], sems[1 - b])
            pltpu.make_async_copy(g_hbm.at[srcb[b]],
                                  bufs[b], sems[b]).wait()
            pltpu.sync_copy(bufs[b], acc_sh.at[dstb[b]], add=True)

        def body(t, carry):
            chunk(2 * t, 0)
            chunk(2 * t + 1, 1)
            return carry

        for h in range(2):
            pltpu.sync_copy(
                packed_hbm.at[pl.ds(pl.multiple_of(row0 + h * hch, 8), maxh)],
                idx_v)
            unpack(0, 0)

            @pl.when(hch > 0)
            def _():
                pltpu.async_copy(g_hbm.at[srcb[0]], buf0, sem0)

            lax.fori_loop(0, hch // 2, body, 0)
        plsc.subcore_barrier()
        pltpu.sync_copy(acc_sh.at[pl.ds(r0, ROWS_PER_TILE)],
                        out_hbm.at[c, pl.ds(r0, ROWS_PER_TILE)])

    return msg_kernel(packed2d, g, zerosf)


_HI = jax.lax.Precision.HIGHEST


def _dinv_g1_body(dega_ref, degb_ref, x_ref, w_ref, dinv_ref, g_ref):
    i = pl.program_id(0)
    deg = dega_ref[...][:, :1] + degb_ref[...][:, :1] + 1.0
    rows = i * BLK + lax.broadcasted_iota(jnp.int32, (BLK, 1), 0)
    dinv = jnp.where(rows < N, lax.rsqrt(deg), 0.0)
    dinvm = jnp.broadcast_to(dinv, (BLK, D))
    dinv_ref[...] = dinvm
    g_ref[...] = jnp.dot(x_ref[...], w_ref[...],
                         preferred_element_type=jnp.float32,
                         precision=_HI) * dinvm


@jax.jit
def _tc_dinv_g1(dega, degb, x_p, W1):
    grid = (NPAD // BLK,)
    blk = pl.BlockSpec((BLK, D), lambda i: (i, 0))
    half = pl.BlockSpec((BLK, HD), lambda i: (i, 0))
    return pl.pallas_call(
        _dinv_g1_body,
        grid=grid,
        in_specs=[half, half, blk, pl.BlockSpec((D, D), lambda i: (0, 0))],
        out_specs=[blk, blk],
        out_shape=[
            jax.ShapeDtypeStruct((NPAD, D), jnp.float32),
            jax.ShapeDtypeStruct((NPAD, D), jnp.float32),
        ],
    )(dega, degb, x_p, W1)


def _layer_body(acca_ref, accb_ref, g_ref, dinv_ref, b_ref, xprev_ref,
                wa_ref, wb_ref, xo_ref, go_ref):
    dm = dinv_ref[...]
    xn = jnp.maximum((acca_ref[...] + accb_ref[...] + g_ref[...]) * dm
                     + b_ref[...], 0.0)
    xo_ref[...] = xn
    go_ref[...] = (jnp.dot(xn, wa_ref[...],
                           preferred_element_type=jnp.float32, precision=_HI)
                   + jnp.dot(xprev_ref[...], wb_ref[...],
                             preferred_element_type=jnp.float32,
                             precision=_HI)) * dm


@jax.jit
def _tc_layer(acca, accb, g, dinvm, b, xprev, Wa, Wb):
    grid = (NPAD // BLK,)
    blk = pl.BlockSpec((BLK, D), lambda i: (i, 0))
    return pl.pallas_call(
        _layer_body,
        grid=grid,
        in_specs=[
            blk, blk, blk, blk,
            pl.BlockSpec((1, D), lambda i: (0, 0)),
            blk,
            pl.BlockSpec((D, D), lambda i: (0, 0)),
            pl.BlockSpec((D, D), lambda i: (0, 0)),
        ],
        out_specs=[blk, blk],
        out_shape=[
            jax.ShapeDtypeStruct((NPAD, D), jnp.float32),
            jax.ShapeDtypeStruct((NPAD, D), jnp.float32),
        ],
    )(acca, accb, g, dinvm, b, xprev, Wa, Wb)


def _final_body(acca_ref, accb_ref, g_ref, dinv_ref, b_ref, o_ref):
    o_ref[...] = ((acca_ref[...] + accb_ref[...] + g_ref[...])
                  * dinv_ref[...] + b_ref[...])


@jax.jit
def _tc_final(acca, accb, g, dinvm, b):
    grid = (NPAD // BLK,)
    blk = pl.BlockSpec((BLK, D), lambda i: (i, 0))
    return pl.pallas_call(
        _final_body,
        grid=grid,
        in_specs=[blk, blk, blk, blk, pl.BlockSpec((1, D), lambda i: (0, 0))],
        out_specs=blk,
        out_shape=jax.ShapeDtypeStruct((NPAD, D), jnp.float32),
    )(acca, accb, g, dinvm, b)


def kernel(x, edge_index, percent, ricci_curvature, W1, b1, W2, b2, W3, b3):
    del percent, ricci_curvature
    E = edge_index.shape[1]
    cpw = 2 * _cdiv(E, NW * CH * 2)
    EPAD = NW * cpw * CH
    total = NW * cpw
    cpw0 = 16 * int(round(total * SPLIT0 / NS / 16))
    cpw1 = total // NS - cpw0

    pad = jnp.full((EPAD - E,), PAD_ROW, dtype=edge_index.dtype)
    src_p = jnp.concatenate([edge_index[0], pad])
    dst_p = jnp.concatenate([edge_index[1], pad]).reshape(NW * cpw, CH)
    packed = (src_p + (dst_p.reshape(-1) << 14)).reshape(NW * cpw, CH)
    ptail = jnp.full((max(cpw0, cpw1), CH), PAD_ROW + (PAD_ROW << 14),
                     dtype=jnp.int32)
    packed2d = jnp.concatenate([packed, ptail], axis=0)
    x_p = jnp.pad(x, ((0, NPAD - N), (0, 0)))
    onesf = jnp.ones((CH, HD), jnp.float32)
    zhalf = jnp.zeros((NPAD, HD), jnp.float32)
    zerosf = jnp.zeros((NPAD, D), jnp.float32)

    degp = _sc_degree(dst_p, zhalf, onesf, cpw=cpw)
    dinvm, g1 = _tc_dinv_g1(degp[0], degp[1], x_p, W1)

    acc1 = _sc_scatter(packed2d, g1, zerosf, cpw0=cpw0, cpw1=cpw1)
    x1, g2 = _tc_layer(acc1[0], acc1[1], g1, dinvm, b1.reshape(1, D),
                       x_p, W2[:D], W2[D:])

    acc2 = _sc_scatter(packed2d, g2, zerosf, cpw0=cpw0, cpw1=cpw1)
    x2, g3 = _tc_layer(acc2[0], acc2[1], g2, dinvm, b2.reshape(1, D),
                       x1, W3[:D], W3[D:])

    acc3 = _sc_scatter(packed2d, g3, zerosf, cpw0=cpw0, cpw1=cpw1)
    out = _tc_final(acc3[0], acc3[1], g3, dinvm, b3.reshape(1, D))

    return out[:N], x1[:N], x2[:N]
